# baseline (speedup 1.0000x reference)
import functools

import jax
import jax.numpy as jnp
from jax import lax
from jax.experimental import pallas as pl
from jax.experimental.pallas import tpu as pltpu
from jax.experimental.pallas import tpu_sc as plsc

KEEP_RATE = 0.8
EPS = 1e-5
NLANE = 16


def _lane_iota():
    return lax.iota(jnp.int32, NLANE)


def _extract0(v):
    return jnp.sum(jnp.where(_lane_iota() == 0, v, 0))


def _rot(v, j):
    idx = (_lane_iota() + j) & (NLANE - 1)
    return v.at[idx].get(mode="promise_in_bounds", unique_indices=True)


def _lane_min(v):
    for j in (1, 2, 4, 8):
        v = jnp.minimum(v, _rot(v, j))
    return _extract0(v)


def _lane_max(v):
    for j in (1, 2, 4, 8):
        v = jnp.maximum(v, _rot(v, j))
    return _extract0(v)


def _ilog2(t):
    hsb = jnp.int32(0)
    for b in (16, 8, 4, 2, 1):
        big = t >= lax.shift_left(jnp.int32(1), b)
        t = jnp.where(big, lax.shift_right_logical(t, b), t)
        hsb = hsb + jnp.where(big, b, 0)
    return hsb


def _sc_topk_body(k, nrows, s, bits_hbm, out_hbm, row_v, idxa_v, valb_v,
                  idxb_v, hist_v, res_v):
    c = lax.axis_index("c")
    sid = lax.axis_index("s")
    wid = sid * 2 + c

    @pl.when(wid < nrows)
    def _():
        r = wid
        pltpu.sync_copy(bits_hbm.at[r], row_v.at[pl.ds(0, s)])

        nvec = s // NLANE

        def mm_step(i, carry):
            mn, mx = carry
            w = row_v[pl.ds(i * NLANE, NLANE)]
            return jnp.minimum(mn, w), jnp.maximum(mx, w)

        big = jnp.full((NLANE,), jnp.int32(0x7FFFFFFF))
        small = jnp.full((NLANE,), jnp.int32(0))
        mnv, mxv = lax.fori_loop(0, nvec, mm_step, (big, small))
        mn = _lane_min(mnv)
        mx = _lane_max(mxv)

        hsb = _ilog2(mx ^ mn)
        start = jnp.maximum(hsb - 4, 0)
        common_mask = (jnp.int32(1) << (start + 5)) - 1
        v_acc = mn & ~common_mask

        def idx_init(i, _):
            idxa_v[pl.ds(i * NLANE, NLANE)] = _lane_iota() + i * NLANE
            return 0

        lax.fori_loop(0, nvec, idx_init, 0)

        n_cur = jnp.int32(s)
        k_rem = jnp.int32(k)

        bufs = [(row_v, idxa_v), (valb_v, idxb_v)]
        ones = jnp.ones((NLANE,), jnp.int32)
        zeros = jnp.zeros((NLANE,), jnp.int32)

        for lvl in range(7):
            src_val, src_idx = bufs[lvl % 2]
            dst_val, dst_idx = bufs[(lvl + 1) % 2]
            shift = jnp.maximum(start - 5 * lvl, 0)

            for b in range(32):
                hist_v[pl.ds(b * NLANE, NLANE)] = zeros

            nv = (n_cur + NLANE - 1) // NLANE

            def hist_step(i, _, src_val=src_val, shift=shift, n_cur=n_cur):
                w = src_val[pl.ds(i * NLANE, NLANE)]
                d = lax.shift_right_logical(w, shift) & 31
                valid = (_lane_iota() + i * NLANE) < n_cur
                plsc.addupdate_scatter(
                    hist_v, [d * NLANE + _lane_iota()], ones, mask=valid)
                return 0

            lax.fori_loop(0, nv, hist_step, 0)

            tot = [jnp.sum(hist_v[pl.ds(b * NLANE, NLANE)])
                   for b in range(32)]
            suf = [None] * 32
            acc = jnp.int32(0)
            for b in range(31, -1, -1):
                acc = acc + tot[b]
                suf[b] = acc
            bsel = jnp.int32(-1)
            for b in range(31, -1, -1):
                bsel = jnp.where((bsel < 0) & (suf[b] >= k_rem),
                                 jnp.int32(b), bsel)
            cnt_gt = jnp.int32(0)
            for b in range(32):
                cnt_gt = cnt_gt + jnp.where(jnp.int32(b) > bsel, tot[b], 0)
            k_rem = k_rem - cnt_gt
            v_acc = v_acc | lax.shift_left(bsel, shift)

            def comp_step(i, off, src_val=src_val, src_idx=src_idx,
                          dst_val=dst_val, dst_idx=dst_idx, shift=shift,
                          n_cur=n_cur, bsel=bsel):
                w = src_val[pl.ds(i * NLANE, NLANE)]
                iv = src_idx[pl.ds(i * NLANE, NLANE)]
                d = lax.shift_right_logical(w, shift) & 31
                valid = (_lane_iota() + i * NLANE) < n_cur
                msk = (d == bsel) & valid
                plsc.store_compressed(dst_val.at[pl.ds(off, NLANE)], w,
                                      mask=msk)
                plsc.store_compressed(dst_idx.at[pl.ds(off, NLANE)], iv,
                                      mask=msk)
                pop = plsc.all_reduce_population_count(msk)
                return off + _extract0(pop)

            n_cur = lax.fori_loop(0, nv, comp_step, jnp.int32(0))

        src_val, src_idx = bufs[7 % 2]
        nv = (n_cur + NLANE - 1) // NLANE

        def cut_step(i, m, src_idx=src_idx, n_cur=n_cur):
            iv = src_idx[pl.ds(i * NLANE, NLANE)]
            pos = _lane_iota() + i * NLANE
            pick = (pos == k_rem - 1) & (pos < n_cur)
            return m + jnp.sum(jnp.where(pick, iv + 1, 0))

        m_cut = lax.fori_loop(0, nv, cut_step, jnp.int32(0))

        lane = _lane_iota()
        res_v[...] = jnp.where(lane == 0, v_acc,
                               jnp.where(lane == 1, m_cut, 0))
        pltpu.sync_copy(res_v, out_hbm.at[r])


def _sc_topk_thresholds(bits, k):
    nrows, s = bits.shape
    mesh = plsc.VectorSubcoreMesh(
        core_axis_name="c", subcore_axis_name="s", num_cores=2,
        num_subcores=16)
    kern = pl.kernel(
        functools.partial(_sc_topk_body, k, nrows, s),
        out_type=jax.ShapeDtypeStruct((nrows, NLANE), jnp.int32),
        mesh=mesh,
        compiler_params=pltpu.CompilerParams(needs_layout_passes=False),
        scratch_types=[
            pltpu.VMEM((s + NLANE,), jnp.int32),
            pltpu.VMEM((s + NLANE,), jnp.int32),
            pltpu.VMEM((s + NLANE,), jnp.int32),
            pltpu.VMEM((s + NLANE,), jnp.int32),
            pltpu.VMEM((32 * NLANE,), jnp.int32),
            pltpu.VMEM((NLANE,), jnp.int32),
        ],
    )
    return kern(bits)


def _scores_body(x_ref, out_ref):
    x = x_ref[0]
    sumsq = jnp.sum(x * x, axis=-1)
    out_ref[0, 0, :] = jnp.sqrt(sumsq)


def _ln_body(nblk, bs, x_ref, v_ref, m_ref, gamma_ref, beta_ref, out_ref):
    i = pl.program_id(0)
    b = i // nblk
    x = x_ref[0]
    d = x.shape[-1]

    sumsq = jnp.sum(x * x, axis=-1, keepdims=True)
    bits = jax.lax.bitcast_convert_type(jnp.sqrt(sumsq), jnp.int32)
    tok = jax.lax.broadcasted_iota(jnp.int32, (bs, 1), 0) + (i % nblk) * bs
    v = v_ref[b]
    m = m_ref[b]
    keep = (bits > v) | ((bits == v) & (tok < m))

    mean = jnp.sum(x, axis=-1, keepdims=True) / d
    cent = x - mean
    var = jnp.sum(cent * cent, axis=-1, keepdims=True) / d
    ln = gamma_ref[...] * cent * jax.lax.rsqrt(var + EPS) + beta_ref[...]
    out_ref[0] = jnp.where(keep, ln, x)


@jax.jit
def kernel(hidden_states, gamma, beta):
    batch, seq, dim = hidden_states.shape
    keep_k = max(1, int(seq * KEEP_RATE))
    bs = min(1024, seq)
    nblk = seq // bs
    grid = batch * nblk

    scores3 = pl.pallas_call(
        _scores_body,
        grid=(grid,),
        in_specs=[
            pl.BlockSpec((1, bs, dim), lambda i: (i // nblk, i % nblk, 0)),
        ],
        out_specs=pl.BlockSpec((1, 1, bs), lambda i: (i, 0, 0)),
        out_shape=jax.ShapeDtypeStruct((grid, 1, bs), jnp.float32),
        compiler_params=pltpu.CompilerParams(
            dimension_semantics=("arbitrary",),
        ),
    )(hidden_states)
    scores = scores3.reshape(batch, seq)

    bits = jax.lax.bitcast_convert_type(scores, jnp.int32)

    vm = _sc_topk_thresholds(bits, keep_k)
    v_arr = vm[:, 0]
    m_arr = vm[:, 1]

    out = pl.pallas_call(
        functools.partial(_ln_body, nblk, bs),
        grid=(grid,),
        in_specs=[
            pl.BlockSpec((1, bs, dim), lambda i: (i // nblk, i % nblk, 0)),
            pl.BlockSpec(memory_space=pltpu.SMEM),
            pl.BlockSpec(memory_space=pltpu.SMEM),
            pl.BlockSpec((dim,), lambda i: (0,)),
            pl.BlockSpec((dim,), lambda i: (0,)),
        ],
        out_specs=pl.BlockSpec((1, bs, dim), lambda i: (i // nblk, i % nblk, 0)),
        out_shape=jax.ShapeDtypeStruct((batch, seq, dim), jnp.float32),
        compiler_params=pltpu.CompilerParams(
            dimension_semantics=("arbitrary",),
        ),
    )(hidden_states, v_arr, m_arr, gamma, beta)
    return out


# speedup vs baseline: 1.0803x; 1.0803x over previous
import functools

import jax
import jax.numpy as jnp
from jax import lax
from jax.experimental import pallas as pl
from jax.experimental.pallas import tpu as pltpu
from jax.experimental.pallas import tpu_sc as plsc

KEEP_RATE = 0.8
EPS = 1e-5
NLANE = 16


def _lane_iota():
    return lax.iota(jnp.int32, NLANE)


def _extract0(v):
    return jnp.sum(jnp.where(_lane_iota() == 0, v, 0))


def _rot(v, j):
    idx = (_lane_iota() + j) & (NLANE - 1)
    return v.at[idx].get(mode="promise_in_bounds", unique_indices=True)


def _lane_min(v):
    for j in (1, 2, 4, 8):
        v = jnp.minimum(v, _rot(v, j))
    return _extract0(v)


def _lane_max(v):
    for j in (1, 2, 4, 8):
        v = jnp.maximum(v, _rot(v, j))
    return _extract0(v)


def _ilog2(t):
    hsb = jnp.int32(0)
    for b in (16, 8, 4, 2, 1):
        big = t >= lax.shift_left(jnp.int32(1), b)
        t = jnp.where(big, lax.shift_right_logical(t, b), t)
        hsb = hsb + jnp.where(big, b, 0)
    return hsb


def _sc_topk_body(k, nrows, s, bits_hbm, out_hbm, row_v, idxa_v, valb_v,
                  idxb_v, hist_v, res_v):
    c = lax.axis_index("c")
    sid = lax.axis_index("s")
    wid = sid * 2 + c

    @pl.when(wid < nrows)
    def _():
        r = wid
        pltpu.sync_copy(bits_hbm.at[r], row_v.at[pl.ds(0, s)])

        nvec = s // NLANE

        def mm_step(i, carry):
            mn, mx = carry
            w = row_v[pl.ds(i * NLANE, NLANE)]
            return jnp.minimum(mn, w), jnp.maximum(mx, w)

        big = jnp.full((NLANE,), jnp.int32(0x7FFFFFFF))
        small = jnp.full((NLANE,), jnp.int32(0))
        mnv, mxv = lax.fori_loop(0, nvec, mm_step, (big, small))
        mn = _lane_min(mnv)
        mx = _lane_max(mxv)

        hsb = _ilog2(mx ^ mn)
        start = jnp.maximum(hsb - 4, 0)
        common_mask = (jnp.int32(1) << (start + 5)) - 1
        v_acc = mn & ~common_mask

        def idx_init(i, _):
            idxa_v[pl.ds(i * NLANE, NLANE)] = _lane_iota() + i * NLANE
            return 0

        lax.fori_loop(0, nvec, idx_init, 0)

        n_cur = jnp.int32(s)
        k_rem = jnp.int32(k)

        bufs = [(row_v, idxa_v), (valb_v, idxb_v)]
        ones = jnp.ones((NLANE,), jnp.int32)
        zeros = jnp.zeros((NLANE,), jnp.int32)

        for lvl in range(7):
            src_val, src_idx = bufs[lvl % 2]
            dst_val, dst_idx = bufs[(lvl + 1) % 2]
            shift = jnp.maximum(start - 5 * lvl, 0)

            for b in range(32):
                hist_v[pl.ds(b * NLANE, NLANE)] = zeros

            nv = (n_cur + NLANE - 1) // NLANE

            def hist_step(i, _, src_val=src_val, shift=shift, n_cur=n_cur):
                w = src_val[pl.ds(i * NLANE, NLANE)]
                d = lax.shift_right_logical(w, shift) & 31
                valid = (_lane_iota() + i * NLANE) < n_cur
                plsc.addupdate_scatter(
                    hist_v, [d * NLANE + _lane_iota()], ones, mask=valid)
                return 0

            lax.fori_loop(0, nv, hist_step, 0)

            tot = [jnp.sum(hist_v[pl.ds(b * NLANE, NLANE)])
                   for b in range(32)]
            suf = [None] * 32
            acc = jnp.int32(0)
            for b in range(31, -1, -1):
                acc = acc + tot[b]
                suf[b] = acc
            bsel = jnp.int32(-1)
            for b in range(31, -1, -1):
                bsel = jnp.where((bsel < 0) & (suf[b] >= k_rem),
                                 jnp.int32(b), bsel)
            cnt_gt = jnp.int32(0)
            for b in range(32):
                cnt_gt = cnt_gt + jnp.where(jnp.int32(b) > bsel, tot[b], 0)
            k_rem = k_rem - cnt_gt
            v_acc = v_acc | lax.shift_left(bsel, shift)

            def comp_step(i, off, src_val=src_val, src_idx=src_idx,
                          dst_val=dst_val, dst_idx=dst_idx, shift=shift,
                          n_cur=n_cur, bsel=bsel):
                w = src_val[pl.ds(i * NLANE, NLANE)]
                iv = src_idx[pl.ds(i * NLANE, NLANE)]
                d = lax.shift_right_logical(w, shift) & 31
                valid = (_lane_iota() + i * NLANE) < n_cur
                msk = (d == bsel) & valid
                plsc.store_compressed(dst_val.at[pl.ds(off, NLANE)], w,
                                      mask=msk)
                plsc.store_compressed(dst_idx.at[pl.ds(off, NLANE)], iv,
                                      mask=msk)
                pop = plsc.all_reduce_population_count(msk)
                return off + _extract0(pop)

            n_cur = lax.fori_loop(0, nv, comp_step, jnp.int32(0))

        src_val, src_idx = bufs[7 % 2]
        nv = (n_cur + NLANE - 1) // NLANE

        def cut_step(i, m, src_idx=src_idx, n_cur=n_cur):
            iv = src_idx[pl.ds(i * NLANE, NLANE)]
            pos = _lane_iota() + i * NLANE
            pick = (pos == k_rem - 1) & (pos < n_cur)
            return m + jnp.sum(jnp.where(pick, iv + 1, 0))

        m_cut = lax.fori_loop(0, nv, cut_step, jnp.int32(0))

        lane = _lane_iota()
        res_v[...] = jnp.where(lane == 0, v_acc,
                               jnp.where(lane == 1, m_cut, 0))
        pltpu.sync_copy(res_v, out_hbm.at[r])


def _sc_topk_thresholds(bits, k):
    nrows, s = bits.shape
    mesh = plsc.VectorSubcoreMesh(
        core_axis_name="c", subcore_axis_name="s", num_cores=2,
        num_subcores=16)
    kern = pl.kernel(
        functools.partial(_sc_topk_body, k, nrows, s),
        out_type=jax.ShapeDtypeStruct((nrows, NLANE), jnp.int32),
        mesh=mesh,
        compiler_params=pltpu.CompilerParams(needs_layout_passes=False),
        scratch_types=[
            pltpu.VMEM((s + NLANE,), jnp.int32),
            pltpu.VMEM((s + NLANE,), jnp.int32),
            pltpu.VMEM((s + NLANE,), jnp.int32),
            pltpu.VMEM((s + NLANE,), jnp.int32),
            pltpu.VMEM((32 * NLANE,), jnp.int32),
            pltpu.VMEM((NLANE,), jnp.int32),
        ],
    )
    return kern(bits)


def _scores_body(x_ref, out_ref):
    x = x_ref[0]
    sumsq = jnp.sum(x * x, axis=-1)
    out_ref[0, 0, :] = jax.lax.bitcast_convert_type(
        jnp.sqrt(sumsq), jnp.int32)


def _ln_body(nblk, bs, x_ref, vm_ref, gamma_ref, beta_ref, out_ref):
    i = pl.program_id(0)
    b = i // nblk
    x = x_ref[0]
    d = x.shape[-1]

    sumsq = jnp.sum(x * x, axis=-1, keepdims=True)
    bits = jax.lax.bitcast_convert_type(jnp.sqrt(sumsq), jnp.int32)
    tok = jax.lax.broadcasted_iota(jnp.int32, (bs, 1), 0) + (i % nblk) * bs
    v = vm_ref[b, 0]
    m = vm_ref[b, 1]
    keep = (bits > v) | ((bits == v) & (tok < m))

    mean = jnp.sum(x, axis=-1, keepdims=True) / d
    cent = x - mean
    var = jnp.sum(cent * cent, axis=-1, keepdims=True) / d
    ln = gamma_ref[...] * cent * jax.lax.rsqrt(var + EPS) + beta_ref[...]
    out_ref[0] = jnp.where(keep, ln, x)


@jax.jit
def kernel(hidden_states, gamma, beta):
    batch, seq, dim = hidden_states.shape
    keep_k = max(1, int(seq * KEEP_RATE))
    bs = min(2048, seq)
    nblk = seq // bs
    grid = batch * nblk

    bits3 = pl.pallas_call(
        _scores_body,
        grid=(grid,),
        in_specs=[
            pl.BlockSpec((1, bs, dim), lambda i: (i // nblk, i % nblk, 0)),
        ],
        out_specs=pl.BlockSpec((1, 1, bs), lambda i: (i, 0, 0)),
        out_shape=jax.ShapeDtypeStruct((grid, 1, bs), jnp.int32),
        compiler_params=pltpu.CompilerParams(
            dimension_semantics=("arbitrary",),
        ),
    )(hidden_states)
    bits = bits3.reshape(batch, seq)

    vm = _sc_topk_thresholds(bits, keep_k)

    out = pl.pallas_call(
        functools.partial(_ln_body, nblk, bs),
        grid=(grid,),
        in_specs=[
            pl.BlockSpec((1, bs, dim), lambda i: (i // nblk, i % nblk, 0)),
            pl.BlockSpec(memory_space=pltpu.SMEM),
            pl.BlockSpec((dim,), lambda i: (0,)),
            pl.BlockSpec((dim,), lambda i: (0,)),
        ],
        out_specs=pl.BlockSpec((1, bs, dim), lambda i: (i // nblk, i % nblk, 0)),
        out_shape=jax.ShapeDtypeStruct((batch, seq, dim), jnp.float32),
        compiler_params=pltpu.CompilerParams(
            dimension_semantics=("arbitrary",),
        ),
    )(hidden_states, vm, gamma, beta)
    return out


# speedup vs baseline: 1.0998x; 1.0180x over previous
import functools

import jax
import jax.numpy as jnp
from jax import lax
from jax.experimental import pallas as pl
from jax.experimental.pallas import tpu as pltpu
from jax.experimental.pallas import tpu_sc as plsc

KEEP_RATE = 0.8
EPS = 1e-5
NLANE = 16


def _lane_iota():
    return lax.iota(jnp.int32, NLANE)


def _extract0(v):
    return jnp.sum(jnp.where(_lane_iota() == 0, v, 0))


def _rot(v, j):
    idx = (_lane_iota() + j) & (NLANE - 1)
    return v.at[idx].get(mode="promise_in_bounds", unique_indices=True)


def _lane_min(v):
    for j in (1, 2, 4, 8):
        v = jnp.minimum(v, _rot(v, j))
    return _extract0(v)


def _lane_max(v):
    for j in (1, 2, 4, 8):
        v = jnp.maximum(v, _rot(v, j))
    return _extract0(v)


def _ilog2(t):
    hsb = jnp.int32(0)
    for b in (16, 8, 4, 2, 1):
        big = t >= lax.shift_left(jnp.int32(1), b)
        t = jnp.where(big, lax.shift_right_logical(t, b), t)
        hsb = hsb + jnp.where(big, b, 0)
    return hsb


def _sc_topk_body(k, nrows, s, bits_hbm, out_hbm, row_v, idxa_v, valb_v,
                  idxb_v, hist_v, res_v):
    c = lax.axis_index("c")
    sid = lax.axis_index("s")
    wid = sid * 2 + c

    @pl.when(wid < nrows)
    def _():
        r = wid
        pltpu.sync_copy(bits_hbm.at[r], row_v.at[pl.ds(0, s)])

        nvec = s // NLANE
        unroll = 4 if nvec % 4 == 0 else 1

        def mm_step(i, carry):
            mn, mx = carry
            for u in range(unroll):
                w = row_v[pl.ds((i * unroll + u) * NLANE, NLANE)]
                mn = jnp.minimum(mn, w)
                mx = jnp.maximum(mx, w)
            return mn, mx

        big = jnp.full((NLANE,), jnp.int32(0x7FFFFFFF))
        small = jnp.full((NLANE,), jnp.int32(0))
        mnv, mxv = lax.fori_loop(0, nvec // unroll, mm_step, (big, small))
        mn = _lane_min(mnv)
        mx = _lane_max(mxv)

        hsb = _ilog2(mx ^ mn)
        start = jnp.maximum(hsb - 4, 0)
        common_mask = (jnp.int32(1) << (start + 5)) - 1
        v_acc = mn & ~common_mask

        ones = jnp.ones((NLANE,), jnp.int32)
        zeros = jnp.zeros((NLANE,), jnp.int32)

        def pick_bucket(k_rem):
            tot = [jnp.sum(hist_v[pl.ds(b * NLANE, NLANE)])
                   for b in range(32)]
            suf = [None] * 32
            acc = jnp.int32(0)
            for b in range(31, -1, -1):
                acc = acc + tot[b]
                suf[b] = acc
            bsel = jnp.int32(-1)
            for b in range(31, -1, -1):
                bsel = jnp.where((bsel < 0) & (suf[b] >= k_rem),
                                 jnp.int32(b), bsel)
            cnt_gt = jnp.int32(0)
            for b in range(32):
                cnt_gt = cnt_gt + jnp.where(jnp.int32(b) > bsel, tot[b], 0)
            return bsel, k_rem - cnt_gt

        for b in range(32):
            hist_v[pl.ds(b * NLANE, NLANE)] = zeros

        def hist0_step(i, _):
            for u in range(unroll):
                w = row_v[pl.ds((i * unroll + u) * NLANE, NLANE)]
                d = lax.shift_right_logical(w, start) & 31
                plsc.addupdate_scatter(
                    hist_v, [d * NLANE + _lane_iota()], ones)
            return 0

        lax.fori_loop(0, nvec // unroll, hist0_step, 0)
        bsel0, k_rem = pick_bucket(jnp.int32(k))
        v_acc = v_acc | lax.shift_left(bsel0, start)

        nq = 4 if nvec % 4 == 0 else 1
        qvec = nvec // nq
        qelem = s // nq

        def comp0_step(i, offs):
            new = []
            for qi in range(nq):
                base = qi * qvec + i
                w = row_v[pl.ds(base * NLANE, NLANE)]
                d = lax.shift_right_logical(w, start) & 31
                msk = d == bsel0
                iv = _lane_iota() + base * NLANE
                dst = qi * qelem + offs[qi]
                plsc.store_compressed(valb_v.at[pl.ds(dst, NLANE)], w,
                                      mask=msk)
                plsc.store_compressed(idxb_v.at[pl.ds(dst, NLANE)], iv,
                                      mask=msk)
                pop = plsc.all_reduce_population_count(msk)
                new.append(offs[qi] + _extract0(pop))
            return tuple(new)

        qcnt = lax.fori_loop(0, qvec, comp0_step,
                             tuple(jnp.int32(0) for _ in range(nq)))

        n_cur = qcnt[0]
        for qi in range(1, nq):
            cnt = qcnt[qi]
            src_base = qi * qelem
            dst_base = n_cur

            def stitch_step(i, _, cnt=cnt, src_base=src_base,
                            dst_base=dst_base):
                msk = (_lane_iota() + i * NLANE) < cnt
                w = valb_v[pl.ds(src_base + i * NLANE, NLANE)]
                iv = idxb_v[pl.ds(src_base + i * NLANE, NLANE)]
                plsc.store_compressed(
                    valb_v.at[pl.ds(dst_base + i * NLANE, NLANE)], w,
                    mask=msk)
                plsc.store_compressed(
                    idxb_v.at[pl.ds(dst_base + i * NLANE, NLANE)], iv,
                    mask=msk)
                return 0

            lax.fori_loop(0, (cnt + NLANE - 1) // NLANE, stitch_step, 0)
            n_cur = n_cur + cnt

        bufs = [(valb_v, idxb_v), (row_v, idxa_v)]

        for lvl in range(1, 7):
            src_val, src_idx = bufs[(lvl - 1) % 2]
            dst_val, dst_idx = bufs[lvl % 2]
            shift = jnp.maximum(start - 5 * lvl, 0)

            for b in range(32):
                hist_v[pl.ds(b * NLANE, NLANE)] = zeros

            nv = (n_cur + NLANE - 1) // NLANE

            def hist_step(i, _, src_val=src_val, shift=shift, n_cur=n_cur):
                w = src_val[pl.ds(i * NLANE, NLANE)]
                d = lax.shift_right_logical(w, shift) & 31
                valid = (_lane_iota() + i * NLANE) < n_cur
                plsc.addupdate_scatter(
                    hist_v, [d * NLANE + _lane_iota()], ones, mask=valid)
                return 0

            lax.fori_loop(0, nv, hist_step, 0)

            bsel, k_rem = pick_bucket(k_rem)
            v_acc = v_acc | lax.shift_left(bsel, shift)

            def comp_step(i, off, src_val=src_val, src_idx=src_idx,
                          dst_val=dst_val, dst_idx=dst_idx, shift=shift,
                          n_cur=n_cur, bsel=bsel):
                w = src_val[pl.ds(i * NLANE, NLANE)]
                iv = src_idx[pl.ds(i * NLANE, NLANE)]
                d = lax.shift_right_logical(w, shift) & 31
                valid = (_lane_iota() + i * NLANE) < n_cur
                msk = (d == bsel) & valid
                plsc.store_compressed(dst_val.at[pl.ds(off, NLANE)], w,
                                      mask=msk)
                plsc.store_compressed(dst_idx.at[pl.ds(off, NLANE)], iv,
                                      mask=msk)
                pop = plsc.all_reduce_population_count(msk)
                return off + _extract0(pop)

            n_cur = lax.fori_loop(0, nv, comp_step, jnp.int32(0))

        src_val, src_idx = bufs[6 % 2]
        nv = (n_cur + NLANE - 1) // NLANE

        def cut_step(i, m, src_idx=src_idx, n_cur=n_cur):
            iv = src_idx[pl.ds(i * NLANE, NLANE)]
            pos = _lane_iota() + i * NLANE
            pick = (pos == k_rem - 1) & (pos < n_cur)
            return m + jnp.sum(jnp.where(pick, iv + 1, 0))

        m_cut = lax.fori_loop(0, nv, cut_step, jnp.int32(0))

        lane = _lane_iota()
        res_v[...] = jnp.where(lane == 0, v_acc,
                               jnp.where(lane == 1, m_cut, 0))
        pltpu.sync_copy(res_v, out_hbm.at[r])


def _sc_topk_thresholds(bits, k):
    nrows, s = bits.shape
    mesh = plsc.VectorSubcoreMesh(
        core_axis_name="c", subcore_axis_name="s", num_cores=2,
        num_subcores=16)
    kern = pl.kernel(
        functools.partial(_sc_topk_body, k, nrows, s),
        out_type=jax.ShapeDtypeStruct((nrows, NLANE), jnp.int32),
        mesh=mesh,
        compiler_params=pltpu.CompilerParams(needs_layout_passes=False),
        scratch_types=[
            pltpu.VMEM((s + NLANE,), jnp.int32),
            pltpu.VMEM((s + NLANE,), jnp.int32),
            pltpu.VMEM((s + NLANE,), jnp.int32),
            pltpu.VMEM((s + NLANE,), jnp.int32),
            pltpu.VMEM((32 * NLANE,), jnp.int32),
            pltpu.VMEM((NLANE,), jnp.int32),
        ],
    )
    return kern(bits)


def _scores_body(x_ref, out_ref):
    x = x_ref[0]
    sumsq = jnp.sum(x * x, axis=-1)
    out_ref[0, 0, :] = jax.lax.bitcast_convert_type(
        jnp.sqrt(sumsq), jnp.int32)


def _ln_body(nblk, bs, x_ref, vm_ref, gamma_ref, beta_ref, out_ref):
    i = pl.program_id(0)
    b = i // nblk
    x = x_ref[0]
    d = x.shape[-1]

    sumsq = jnp.sum(x * x, axis=-1, keepdims=True)
    bits = jax.lax.bitcast_convert_type(jnp.sqrt(sumsq), jnp.int32)
    tok = jax.lax.broadcasted_iota(jnp.int32, (bs, 1), 0) + (i % nblk) * bs
    v = vm_ref[b, 0]
    m = vm_ref[b, 1]
    keep = (bits > v) | ((bits == v) & (tok < m))

    mean = jnp.sum(x, axis=-1, keepdims=True) / d
    cent = x - mean
    var = jnp.sum(cent * cent, axis=-1, keepdims=True) / d
    ln = gamma_ref[...] * cent * jax.lax.rsqrt(var + EPS) + beta_ref[...]
    out_ref[0] = jnp.where(keep, ln, x)


@jax.jit
def kernel(hidden_states, gamma, beta):
    batch, seq, dim = hidden_states.shape
    keep_k = max(1, int(seq * KEEP_RATE))
    bs = min(2048, seq)
    nblk = seq // bs
    grid = batch * nblk

    bits3 = pl.pallas_call(
        _scores_body,
        grid=(grid,),
        in_specs=[
            pl.BlockSpec((1, bs, dim), lambda i: (i // nblk, i % nblk, 0)),
        ],
        out_specs=pl.BlockSpec((1, 1, bs), lambda i: (i, 0, 0)),
        out_shape=jax.ShapeDtypeStruct((grid, 1, bs), jnp.int32),
        compiler_params=pltpu.CompilerParams(
            dimension_semantics=("arbitrary",),
        ),
    )(hidden_states)
    bits = bits3.reshape(batch, seq)

    vm = _sc_topk_thresholds(bits, keep_k)

    out = pl.pallas_call(
        functools.partial(_ln_body, nblk, bs),
        grid=(grid,),
        in_specs=[
            pl.BlockSpec((1, bs, dim), lambda i: (i // nblk, i % nblk, 0)),
            pl.BlockSpec(memory_space=pltpu.SMEM),
            pl.BlockSpec((dim,), lambda i: (0,)),
            pl.BlockSpec((dim,), lambda i: (0,)),
        ],
        out_specs=pl.BlockSpec((1, bs, dim), lambda i: (i // nblk, i % nblk, 0)),
        out_shape=jax.ShapeDtypeStruct((batch, seq, dim), jnp.float32),
        compiler_params=pltpu.CompilerParams(
            dimension_semantics=("arbitrary",),
        ),
    )(hidden_states, vm, gamma, beta)
    return out


# speedup vs baseline: 1.1695x; 1.0634x over previous
import functools

import jax
import jax.numpy as jnp
from jax import lax
from jax.experimental import pallas as pl
from jax.experimental.pallas import tpu as pltpu
from jax.experimental.pallas import tpu_sc as plsc

KEEP_RATE = 0.8
EPS = 1e-5
NLANE = 16


def _lane_iota():
    return lax.iota(jnp.int32, NLANE)


def _extract0(v):
    return jnp.sum(jnp.where(_lane_iota() == 0, v, 0))


def _rot(v, j):
    idx = (_lane_iota() + j) & (NLANE - 1)
    return v.at[idx].get(mode="promise_in_bounds", unique_indices=True)


def _lane_min(v):
    for j in (1, 2, 4, 8):
        v = jnp.minimum(v, _rot(v, j))
    return _extract0(v)


def _lane_max(v):
    for j in (1, 2, 4, 8):
        v = jnp.maximum(v, _rot(v, j))
    return _extract0(v)


def _ilog2(t):
    hsb = jnp.int32(0)
    for b in (16, 8, 4, 2, 1):
        big = t >= lax.shift_left(jnp.int32(1), b)
        t = jnp.where(big, lax.shift_right_logical(t, b), t)
        hsb = hsb + jnp.where(big, b, 0)
    return hsb


def _sc_topk_body(k, nrows, s, bits_hbm, out_hbm, row_v, idxa_v, valb_v,
                  idxb_v, hist_v, res_v):
    c = lax.axis_index("c")
    sid = lax.axis_index("s")
    wid = sid * 2 + c

    @pl.when(wid < nrows)
    def _():
        r = wid
        pltpu.sync_copy(bits_hbm.at[r], row_v.at[pl.ds(0, s)])

        nvec = s // NLANE
        unroll = 4 if nvec % 4 == 0 else 1

        def mm_step(i, carry):
            mn, mx = carry
            w = row_v[pl.ds(i * NLANE, NLANE)]
            return jnp.minimum(mn, w), jnp.maximum(mx, w)

        big = jnp.full((NLANE,), jnp.int32(0x7FFFFFFF))
        small = jnp.full((NLANE,), jnp.int32(0))
        mnv, mxv = plsc.parallel_loop(
            0, nvec, carry=(big, small), unroll=unroll)(mm_step)
        mn = _lane_min(mnv)
        mx = _lane_max(mxv)

        hsb = _ilog2(mx ^ mn)
        start = jnp.maximum(hsb - 4, 0)
        common_mask = (jnp.int32(1) << (start + 5)) - 1
        v_acc = mn & ~common_mask

        ones = jnp.ones((NLANE,), jnp.int32)
        zeros = jnp.zeros((NLANE,), jnp.int32)

        def pick_bucket(k_rem):
            tot = [jnp.sum(hist_v[pl.ds(b * NLANE, NLANE)])
                   for b in range(32)]
            suf = [None] * 32
            acc = jnp.int32(0)
            for b in range(31, -1, -1):
                acc = acc + tot[b]
                suf[b] = acc
            bsel = jnp.int32(-1)
            for b in range(31, -1, -1):
                bsel = jnp.where((bsel < 0) & (suf[b] >= k_rem),
                                 jnp.int32(b), bsel)
            cnt_gt = jnp.int32(0)
            for b in range(32):
                cnt_gt = cnt_gt + jnp.where(jnp.int32(b) > bsel, tot[b], 0)
            return bsel, k_rem - cnt_gt

        for b in range(32):
            hist_v[pl.ds(b * NLANE, NLANE)] = zeros

        def hist0_step(i):
            w = row_v[pl.ds(i * NLANE, NLANE)]
            d = lax.shift_right_logical(w, start) & 31
            plsc.addupdate_scatter(
                hist_v, [d * NLANE + _lane_iota()], ones)

        plsc.parallel_loop(0, nvec, unroll=unroll)(hist0_step)
        bsel0, k_rem = pick_bucket(jnp.int32(k))
        v_acc = v_acc | lax.shift_left(bsel0, start)

        nq = 4 if nvec % 4 == 0 else 1
        qvec = nvec // nq
        qelem = s // nq

        def comp0_step(i, offs):
            new = []
            for qi in range(nq):
                base = qi * qvec + i
                w = row_v[pl.ds(base * NLANE, NLANE)]
                d = lax.shift_right_logical(w, start) & 31
                msk = d == bsel0
                iv = _lane_iota() + base * NLANE
                dst = qi * qelem + offs[qi]
                plsc.store_compressed(valb_v.at[pl.ds(dst, NLANE)], w,
                                      mask=msk)
                plsc.store_compressed(idxb_v.at[pl.ds(dst, NLANE)], iv,
                                      mask=msk)
                pop = plsc.all_reduce_population_count(msk)
                new.append(offs[qi] + _extract0(pop))
            return tuple(new)

        qcnt = plsc.parallel_loop(
            0, qvec, carry=tuple(jnp.int32(0) for _ in range(nq)))(
                lambda i, offs: comp0_step(i, offs))

        n_cur = qcnt[0]
        for qi in range(1, nq):
            cnt = qcnt[qi]
            src_base = qi * qelem
            dst_base = n_cur

            def stitch_step(i, cnt=cnt, src_base=src_base,
                            dst_base=dst_base):
                msk = (_lane_iota() + i * NLANE) < cnt
                w = valb_v[pl.ds(src_base + i * NLANE, NLANE)]
                iv = idxb_v[pl.ds(src_base + i * NLANE, NLANE)]
                plsc.store_compressed(
                    valb_v.at[pl.ds(dst_base + i * NLANE, NLANE)], w,
                    mask=msk)
                plsc.store_compressed(
                    idxb_v.at[pl.ds(dst_base + i * NLANE, NLANE)], iv,
                    mask=msk)

            plsc.parallel_loop(0, (cnt + NLANE - 1) // NLANE)(stitch_step)
            n_cur = n_cur + cnt

        bufs = [(valb_v, idxb_v), (row_v, idxa_v)]

        for lvl in range(1, 7):
            src_val, src_idx = bufs[(lvl - 1) % 2]
            dst_val, dst_idx = bufs[lvl % 2]
            shift = jnp.maximum(start - 5 * lvl, 0)

            for b in range(32):
                hist_v[pl.ds(b * NLANE, NLANE)] = zeros

            nv = (n_cur + NLANE - 1) // NLANE

            def hist_step(i, src_val=src_val, shift=shift, n_cur=n_cur):
                w = src_val[pl.ds(i * NLANE, NLANE)]
                d = lax.shift_right_logical(w, shift) & 31
                valid = (_lane_iota() + i * NLANE) < n_cur
                plsc.addupdate_scatter(
                    hist_v, [d * NLANE + _lane_iota()], ones, mask=valid)

            plsc.parallel_loop(0, nv)(hist_step)

            bsel, k_rem = pick_bucket(k_rem)
            v_acc = v_acc | lax.shift_left(bsel, shift)

            def comp_step(i, off, src_val=src_val, src_idx=src_idx,
                          dst_val=dst_val, dst_idx=dst_idx, shift=shift,
                          n_cur=n_cur, bsel=bsel):
                w = src_val[pl.ds(i * NLANE, NLANE)]
                iv = src_idx[pl.ds(i * NLANE, NLANE)]
                d = lax.shift_right_logical(w, shift) & 31
                valid = (_lane_iota() + i * NLANE) < n_cur
                msk = (d == bsel) & valid
                plsc.store_compressed(dst_val.at[pl.ds(off, NLANE)], w,
                                      mask=msk)
                plsc.store_compressed(dst_idx.at[pl.ds(off, NLANE)], iv,
                                      mask=msk)
                pop = plsc.all_reduce_population_count(msk)
                return off + _extract0(pop)

            n_cur = plsc.parallel_loop(
                0, nv, carry=jnp.int32(0))(comp_step)

        src_val, src_idx = bufs[6 % 2]
        nv = (n_cur + NLANE - 1) // NLANE

        def cut_step(i, m, src_idx=src_idx, n_cur=n_cur):
            iv = src_idx[pl.ds(i * NLANE, NLANE)]
            pos = _lane_iota() + i * NLANE
            pick = (pos == k_rem - 1) & (pos < n_cur)
            return m + jnp.sum(jnp.where(pick, iv + 1, 0))

        m_cut = plsc.parallel_loop(0, nv, carry=jnp.int32(0))(cut_step)

        lane = _lane_iota()
        res_v[...] = jnp.where(lane == 0, v_acc,
                               jnp.where(lane == 1, m_cut, 0))
        pltpu.sync_copy(res_v, out_hbm.at[r])


def _sc_topk_thresholds(bits, k):
    nrows, s = bits.shape
    mesh = plsc.VectorSubcoreMesh(
        core_axis_name="c", subcore_axis_name="s", num_cores=2,
        num_subcores=16)
    kern = pl.kernel(
        functools.partial(_sc_topk_body, k, nrows, s),
        out_type=jax.ShapeDtypeStruct((nrows, NLANE), jnp.int32),
        mesh=mesh,
        compiler_params=pltpu.CompilerParams(needs_layout_passes=False),
        scratch_types=[
            pltpu.VMEM((s + NLANE,), jnp.int32),
            pltpu.VMEM((s + NLANE,), jnp.int32),
            pltpu.VMEM((s + NLANE,), jnp.int32),
            pltpu.VMEM((s + NLANE,), jnp.int32),
            pltpu.VMEM((32 * NLANE,), jnp.int32),
            pltpu.VMEM((NLANE,), jnp.int32),
        ],
    )
    return kern(bits)


def _scores_body(x_ref, out_ref):
    x = x_ref[0]
    sumsq = jnp.sum(x * x, axis=-1)
    out_ref[0, 0, :] = jax.lax.bitcast_convert_type(
        jnp.sqrt(sumsq), jnp.int32)


def _ln_body(nblk, bs, x_ref, vm_ref, gamma_ref, beta_ref, out_ref):
    i = pl.program_id(0)
    b = i // nblk
    x = x_ref[0]
    d = x.shape[-1]

    sumsq = jnp.sum(x * x, axis=-1, keepdims=True)
    bits = jax.lax.bitcast_convert_type(jnp.sqrt(sumsq), jnp.int32)
    tok = jax.lax.broadcasted_iota(jnp.int32, (bs, 1), 0) + (i % nblk) * bs
    v = vm_ref[b, 0]
    m = vm_ref[b, 1]
    keep = (bits > v) | ((bits == v) & (tok < m))

    mean = jnp.sum(x, axis=-1, keepdims=True) / d
    cent = x - mean
    var = jnp.sum(cent * cent, axis=-1, keepdims=True) / d
    ln = gamma_ref[...] * cent * jax.lax.rsqrt(var + EPS) + beta_ref[...]
    out_ref[0] = jnp.where(keep, ln, x)


@jax.jit
def kernel(hidden_states, gamma, beta):
    batch, seq, dim = hidden_states.shape
    keep_k = max(1, int(seq * KEEP_RATE))
    bs = min(2048, seq)
    nblk = seq // bs
    grid = batch * nblk

    bits3 = pl.pallas_call(
        _scores_body,
        grid=(grid,),
        in_specs=[
            pl.BlockSpec((1, bs, dim), lambda i: (i // nblk, i % nblk, 0)),
        ],
        out_specs=pl.BlockSpec((1, 1, bs), lambda i: (i, 0, 0)),
        out_shape=jax.ShapeDtypeStruct((grid, 1, bs), jnp.int32),
        compiler_params=pltpu.CompilerParams(
            dimension_semantics=("arbitrary",),
        ),
    )(hidden_states)
    bits = bits3.reshape(batch, seq)

    vm = _sc_topk_thresholds(bits, keep_k)

    out = pl.pallas_call(
        functools.partial(_ln_body, nblk, bs),
        grid=(grid,),
        in_specs=[
            pl.BlockSpec((1, bs, dim), lambda i: (i // nblk, i % nblk, 0)),
            pl.BlockSpec(memory_space=pltpu.SMEM),
            pl.BlockSpec((dim,), lambda i: (0,)),
            pl.BlockSpec((dim,), lambda i: (0,)),
        ],
        out_specs=pl.BlockSpec((1, bs, dim), lambda i: (i // nblk, i % nblk, 0)),
        out_shape=jax.ShapeDtypeStruct((batch, seq, dim), jnp.float32),
        compiler_params=pltpu.CompilerParams(
            dimension_semantics=("arbitrary",),
        ),
    )(hidden_states, vm, gamma, beta)
    return out
